# baseline (speedup 1.0000x reference)
import functools

import jax
import jax.numpy as jnp
from jax import lax
from jax.experimental import pallas as pl
from jax.experimental.pallas import tpu as pltpu
from jax.experimental.pallas import tpu_sc as plsc

NUM_EMB = 1000000
B = 1024
L = 200
D = 50
NW = 32
TOTAL = B * L
PER_WORKER = TOTAL // NW
CHUNK = 128
NCHUNK = PER_WORKER // CHUNK
NPAIR = NCHUNK // 2
GPR = 4
NSTREAM = CHUNK * GPR // 128


def _sc_body(idx_hbm, t1_hbm, t2_hbm, out_hbm, idx_v, gidx, gbuf1, gbuf2,
             obuf, sem):
    wid = lax.axis_index("s") * 2 + lax.axis_index("c")
    pltpu.sync_copy(idx_hbm.at[wid], idx_v)
    out_base = wid * PER_WORKER

    iota = lax.iota(jnp.int32, 16)
    kvec = iota & 3
    lsub = iota >> 2

    def build_gidx(j, slot):
        for m in range(NSTREAM):
            for u in range(8):
                addr = j * CHUNK + 32 * m + 4 * u + lsub
                idxs = plsc.load_gather(idx_v, [addr])
                val = ((idxs * 25) >> 3) + kvec
                gidx.at[slot, m][pl.ds(16 * u, 16)] = val

    def fire(slot):
        copies = []
        for m in range(NSTREAM):
            dst = pl.ds(128 * m, 128)
            copies.append(
                pltpu.async_copy(t1_hbm.at[gidx.at[slot, m]],
                                 gbuf1.at[slot].at[dst], sem))
            copies.append(
                pltpu.async_copy(t2_hbm.at[gidx.at[slot, m]],
                                 gbuf2.at[slot].at[dst], sem))
        return copies

    def realign_store(j, slot):
        for t in range(8):
            lanes = 16 * t + iota
            idxs = plsc.load_gather(idx_v, [j * CHUNK + lanes])
            a = (idxs << 1) & 15
            srow_base = lanes * 4
            c1 = lanes * 0
            c2 = c1 + D
            for c in range(D):
                srow = srow_base + (a >> 4)
                scol = a & 15
                v1 = plsc.load_gather(gbuf1.at[slot], [srow, scol])
                v2 = plsc.load_gather(gbuf2.at[slot], [srow, scol])
                plsc.store_scatter(obuf, [lanes, c1], v1)
                plsc.store_scatter(obuf, [lanes, c2], v2)
                a = a + 1
                c1 = c1 + 1
                c2 = c2 + 1
        pltpu.sync_copy(obuf,
                        out_hbm.at[pl.ds(out_base + j * CHUNK, CHUNK)])

    def pair_step(jj, carry):
        j0 = 2 * jj
        j1 = j0 + 1
        build_gidx(j0, 0)
        build_gidx(j1, 1)
        c0 = fire(0)
        c1 = fire(1)
        for c in c0:
            c.wait()
        realign_store(j0, 0)
        for c in c1:
            c.wait()
        realign_store(j1, 1)
        return carry

    lax.fori_loop(0, NPAIR, pair_step, 0)


@jax.jit
def _lookup(idx2, t1g, t2g):
    mesh = plsc.VectorSubcoreMesh(core_axis_name="c", subcore_axis_name="s")
    return pl.kernel(
        _sc_body,
        out_type=jax.ShapeDtypeStruct((TOTAL, 2 * D), jnp.float32),
        mesh=mesh,
        scratch_types=[
            pltpu.VMEM((PER_WORKER,), jnp.int32),
            pltpu.VMEM((2, NSTREAM, 128), jnp.int32),
            pltpu.VMEM((2, CHUNK * GPR, 16), jnp.float32),
            pltpu.VMEM((2, CHUNK * GPR, 16), jnp.float32),
            pltpu.VMEM((CHUNK, 2 * D), jnp.float32),
            pltpu.SemaphoreType.DMA,
        ],
        compiler_params=pltpu.CompilerParams(use_tc_tiling_on_sc=False,
                                             needs_layout_passes=False),
    )(idx2, t1g, t2g)


BL = 2048
TGRID = (NUM_EMB + BL - 1) // BL


def _tbody(in_ref, out_ref):
    out_ref[...] = in_ref[...].T


def _transpose_tc(wt):
    return pl.pallas_call(
        _tbody,
        grid=(TGRID,),
        in_specs=[pl.BlockSpec((D, BL), lambda i: (0, i))],
        out_specs=pl.BlockSpec((BL, D), lambda i: (i, 0)),
        out_shape=jax.ShapeDtypeStruct((NUM_EMB, D), jnp.float32),
    )(wt)


def kernel(indices, weight1, weight2):
    idx2 = indices.reshape(NW, PER_WORKER)
    t1g = _transpose_tc(weight1.T).reshape(-1, 16)
    t2g = _transpose_tc(weight2.T).reshape(-1, 16)
    out = _lookup(idx2, t1g, t2g)
    return out.reshape(B, L, 2 * D)


# speedup vs baseline: 1.0258x; 1.0258x over previous
import functools

import jax
import jax.numpy as jnp
from jax import lax
from jax.experimental import pallas as pl
from jax.experimental.pallas import tpu as pltpu
from jax.experimental.pallas import tpu_sc as plsc

NUM_EMB = 1000000
B = 1024
L = 200
D = 50
NW = 32
TOTAL = B * L
PER_WORKER = TOTAL // NW
CHUNK = 128
NCHUNK = PER_WORKER // CHUNK
GPR = 2
NSTREAM = CHUNK * GPR // 128
GW = 64
MAXROW = NUM_EMB * D // GW - 1


def _sc_body(idx_hbm, t1_hbm, t2_hbm, out_hbm, idx_v, gidx, gbuf1, gbuf2,
             obuf, sem):
    wid = lax.axis_index("s") * 2 + lax.axis_index("c")
    pltpu.sync_copy(idx_hbm.at[wid], idx_v)
    out_base = wid * PER_WORKER

    iota = lax.iota(jnp.int32, 16)
    kvec = iota & 1
    lsub = iota >> 1

    def chunk_step(j, carry):
        for m in range(NSTREAM):
            for u in range(8):
                addr = j * CHUNK + 64 * m + 8 * u + lsub
                idxs = plsc.load_gather(idx_v, [addr])
                val = jnp.minimum(((idxs * 25) >> 5) + kvec, MAXROW)
                gidx.at[m][pl.ds(16 * u, 16)] = val
        copies = []
        for m in range(NSTREAM):
            dst = pl.ds(128 * m, 128)
            copies.append(
                pltpu.async_copy(t1_hbm.at[gidx.at[m]], gbuf1.at[dst], sem))
            copies.append(
                pltpu.async_copy(t2_hbm.at[gidx.at[m]], gbuf2.at[dst], sem))
        for c in copies:
            c.wait()

        for t in range(8):
            lanes = 16 * t + iota
            idxs = plsc.load_gather(idx_v, [j * CHUNK + lanes])
            a = (idxs * 50) & 63
            srow_base = lanes * 2
            c1 = lanes * 0
            c2 = c1 + D
            for c in range(D):
                srow = srow_base + (a >> 6)
                scol = a & 63
                v1 = plsc.load_gather(gbuf1, [srow, scol])
                v2 = plsc.load_gather(gbuf2, [srow, scol])
                plsc.store_scatter(obuf, [lanes, c1], v1)
                plsc.store_scatter(obuf, [lanes, c2], v2)
                a = a + 1
                c1 = c1 + 1
                c2 = c2 + 1

        pltpu.sync_copy(obuf,
                        out_hbm.at[pl.ds(out_base + j * CHUNK, CHUNK)])
        return carry

    lax.fori_loop(0, NCHUNK, chunk_step, 0)


@jax.jit
def _lookup(idx2, t1g, t2g):
    mesh = plsc.VectorSubcoreMesh(core_axis_name="c", subcore_axis_name="s")
    return pl.kernel(
        _sc_body,
        out_type=jax.ShapeDtypeStruct((TOTAL, 2 * D), jnp.float32),
        mesh=mesh,
        scratch_types=[
            pltpu.VMEM((PER_WORKER,), jnp.int32),
            pltpu.VMEM((NSTREAM, 128), jnp.int32),
            pltpu.VMEM((CHUNK * GPR, GW), jnp.float32),
            pltpu.VMEM((CHUNK * GPR, GW), jnp.float32),
            pltpu.VMEM((CHUNK, 2 * D), jnp.float32),
            pltpu.SemaphoreType.DMA,
        ],
        compiler_params=pltpu.CompilerParams(use_tc_tiling_on_sc=False,
                                             needs_layout_passes=False),
    )(idx2, t1g, t2g)


BL = 2048
TGRID = (NUM_EMB + BL - 1) // BL


def _tbody(in_ref, out_ref):
    out_ref[...] = in_ref[...].T


def _transpose_tc(wt):
    return pl.pallas_call(
        _tbody,
        grid=(TGRID,),
        in_specs=[pl.BlockSpec((D, BL), lambda i: (0, i))],
        out_specs=pl.BlockSpec((BL, D), lambda i: (i, 0)),
        out_shape=jax.ShapeDtypeStruct((NUM_EMB, D), jnp.float32),
    )(wt)


def kernel(indices, weight1, weight2):
    idx2 = indices.reshape(NW, PER_WORKER)
    t1g = _transpose_tc(weight1.T).reshape(-1, GW)
    t2g = _transpose_tc(weight2.T).reshape(-1, GW)
    out = _lookup(idx2, t1g, t2g)
    return out.reshape(B, L, 2 * D)
